# baseline (speedup 1.0000x reference)
import functools

import jax
import jax.numpy as jnp
from jax import lax
from jax.experimental import pallas as pl
from jax.experimental.pallas import tpu as pltpu
from jax.experimental.pallas import tpu_sc as plsc

N = 10000
F = 128
H = 64
G = 64

N_PAD = 10240
BLK = 512
CHUNK = 128
CW = 16
NC = 2
NS = 16
NW = NC * NS
RPT = N_PAD // NS


NBUF = 2


def _seg_body_l1(nchunks, p2_hbm, src_hbm, dst_hbm, zh, z16, ones_hbm,
                 out_acc, out_cnt,
                 idx_s, idx_d, rows_v, ones_v, acc_sh, cnt_sh, gsem, ssem, csem):
    cid = lax.axis_index("c")
    sid = lax.axis_index("s")
    base = sid * RPT
    chalf = nchunks // 2
    pltpu.sync_copy(zh.at[pl.ds(base, RPT)], acc_sh.at[pl.ds(base, RPT)])
    pltpu.sync_copy(z16.at[pl.ds(base, RPT)], cnt_sh.at[pl.ds(base, RPT)])
    pltpu.sync_copy(src_hbm.at[sid], idx_s)
    pltpu.sync_copy(dst_hbm.at[sid], idx_d)
    pltpu.sync_copy(ones_hbm, ones_v)
    plsc.subcore_barrier()

    ptab = p2_hbm.at[cid]
    pltpu.async_copy(ptab.at[idx_s.at[0]], rows_v.at[0], gsem.at[0])

    def group(g, carry):
        for b in range(NBUF):
            j = g * NBUF + b
            ob = 1 - b

            @pl.when(j >= 1)
            def _():
                pltpu.make_async_copy(rows_v.at[ob], acc_sh.at[idx_d.at[0]],
                                      ssem.at[ob]).wait()

            @pl.when(j + 1 < nchunks)
            def _():
                pltpu.async_copy(ptab.at[idx_s.at[j + 1]], rows_v.at[ob],
                                 gsem.at[ob])

            pltpu.make_async_copy(ptab.at[idx_s.at[j]], rows_v.at[b],
                                  gsem.at[b]).wait()
            pltpu.async_copy(rows_v.at[b], acc_sh.at[idx_d.at[j]],
                             ssem.at[b], add=True)
            mine = ((cid == 0) & (j < chalf)) | ((cid == 1) & (j >= chalf))

            @pl.when(mine)
            def _():
                pltpu.async_copy(ones_v, cnt_sh.at[idx_d.at[j]], csem, add=True)
        return carry

    lax.fori_loop(0, nchunks // NBUF, group, 0)
    b_last = (nchunks - 1) % NBUF
    pltpu.make_async_copy(rows_v.at[b_last], acc_sh.at[idx_d.at[0]],
                          ssem.at[b_last]).wait()

    def drain(j, carry):
        pltpu.make_async_copy(ones_v, cnt_sh.at[idx_d.at[0]], csem).wait()
        return carry

    lax.fori_loop(0, chalf, drain, 0)
    plsc.subcore_barrier()
    pltpu.sync_copy(acc_sh.at[pl.ds(base, RPT)], out_acc.at[cid, pl.ds(base, RPT)])
    pltpu.sync_copy(cnt_sh.at[pl.ds(base, RPT)], out_cnt.at[cid, pl.ds(base, RPT)])


def _seg_body_nocnt(nchunks, p_hbm, src_hbm, dst_hbm, z64,
                    out_acc,
                    idx_s, idx_d, rows_v, acc_sh, gsem, ssem):
    cid = lax.axis_index("c")
    sid = lax.axis_index("s")
    wid = sid * NC + cid
    base = sid * RPT
    pltpu.sync_copy(z64.at[pl.ds(base, RPT)], acc_sh.at[pl.ds(base, RPT)])
    pltpu.sync_copy(src_hbm.at[wid], idx_s)
    pltpu.sync_copy(dst_hbm.at[wid], idx_d)
    plsc.subcore_barrier()

    ptab = p_hbm.at[cid]
    pltpu.async_copy(ptab.at[idx_s.at[0]], rows_v.at[0], gsem.at[0])

    def group(g, carry):
        for b in range(NBUF):
            j = g * NBUF + b
            ob = 1 - b

            @pl.when(j >= 1)
            def _():
                pltpu.make_async_copy(rows_v.at[ob], acc_sh.at[idx_d.at[j - 1]],
                                      ssem.at[ob]).wait()

            @pl.when(j + 1 < nchunks)
            def _():
                pltpu.async_copy(ptab.at[idx_s.at[j + 1]], rows_v.at[ob],
                                 gsem.at[ob])

            pltpu.make_async_copy(ptab.at[idx_s.at[j]], rows_v.at[b],
                                  gsem.at[b]).wait()
            pltpu.async_copy(rows_v.at[b], acc_sh.at[idx_d.at[j]],
                             ssem.at[b], add=True)
        return carry

    lax.fori_loop(0, nchunks // NBUF, group, 0)
    b_last = (nchunks - 1) % NBUF
    pltpu.make_async_copy(rows_v.at[b_last], acc_sh.at[idx_d.at[nchunks - 1]],
                          ssem.at[b_last]).wait()
    plsc.subcore_barrier()
    pltpu.sync_copy(acc_sh.at[pl.ds(base, RPT)], out_acc.at[cid, pl.ds(base, RPT)])


def _seg_sum_l1(x2, src2, dst2, zh, z16, ones, nchunks):
    mesh = plsc.VectorSubcoreMesh(core_axis_name="c", subcore_axis_name="s")
    kern = pl.kernel(
        functools.partial(_seg_body_l1, nchunks),
        mesh=mesh,
        compiler_params=pltpu.CompilerParams(use_tc_tiling_on_sc=False),
        out_type=[
            jax.ShapeDtypeStruct((NC, N_PAD, H), jnp.float32),
            jax.ShapeDtypeStruct((NC, N_PAD, CW), jnp.float32),
        ],
        scratch_types=[
            pltpu.VMEM((nchunks, CHUNK), jnp.int32),
            pltpu.VMEM((nchunks, CHUNK), jnp.int32),
            pltpu.VMEM((NBUF, CHUNK, H), jnp.float32),
            pltpu.VMEM((CHUNK, CW), jnp.float32),
            pltpu.VMEM_SHARED((N_PAD, H), jnp.float32),
            pltpu.VMEM_SHARED((N_PAD, CW), jnp.float32),
            pltpu.SemaphoreType.DMA((NBUF,)),
            pltpu.SemaphoreType.DMA((NBUF,)),
            pltpu.SemaphoreType.DMA,
        ],
    )
    return kern(x2, src2, dst2, zh, z16, ones)


def _seg_sum_nocnt(p, src3, dst3, zacc, nchunks, width):
    mesh = plsc.VectorSubcoreMesh(core_axis_name="c", subcore_axis_name="s")
    kern = pl.kernel(
        functools.partial(_seg_body_nocnt, nchunks),
        mesh=mesh,
        compiler_params=pltpu.CompilerParams(use_tc_tiling_on_sc=False),
        out_type=jax.ShapeDtypeStruct((NC, N_PAD, width), jnp.float32),
        scratch_types=[
            pltpu.VMEM((nchunks, CHUNK), jnp.int32),
            pltpu.VMEM((nchunks, CHUNK), jnp.int32),
            pltpu.VMEM((NBUF, CHUNK, width), jnp.float32),
            pltpu.VMEM_SHARED((N_PAD, width), jnp.float32),
            pltpu.SemaphoreType.DMA((NBUF,)),
            pltpu.SemaphoreType.DMA((NBUF,)),
        ],
    )
    return kern(p, src3, dst3, zacc)


def _layer1_body(acc_ref, cnt_ref, x_ref, wl_ref, wr_ref, b_ref, h_ref):
    a = jnp.concatenate([acc_ref[0], acc_ref[1]], axis=1)
    c = cnt_ref[0, :, :1] + cnt_ref[1, :, :1]
    mean = a / jnp.maximum(c, 1.0)
    h = jnp.maximum(
        jnp.dot(mean, wl_ref[...], preferred_element_type=jnp.float32)
        + b_ref[...]
        + jnp.dot(x_ref[...], wr_ref[...], preferred_element_type=jnp.float32),
        0.0)
    h_ref[0] = h
    h_ref[1] = h


def _layer1(acc, cnt, x_p, wlT, wrT, b2d):
    return pl.pallas_call(
        _layer1_body,
        grid=(N_PAD // BLK,),
        in_specs=[
            pl.BlockSpec((NC, BLK, H), lambda i: (0, i, 0)),
            pl.BlockSpec((NC, BLK, CW), lambda i: (0, i, 0)),
            pl.BlockSpec((BLK, F), lambda i: (i, 0)),
            pl.BlockSpec((F, H), lambda i: (0, 0)),
            pl.BlockSpec((F, H), lambda i: (0, 0)),
            pl.BlockSpec((1, H), lambda i: (0, 0)),
        ],
        out_specs=pl.BlockSpec((NC, BLK, H), lambda i: (0, i, 0)),
        out_shape=jax.ShapeDtypeStruct((NC, N_PAD, H), jnp.float32),
    )(acc, cnt, x_p, wlT, wrT, b2d)


def _final_body(acc_ref, cnt_ref, h1_ref, wl_ref, wr_ref, b_ref, batch_ref,
                wa1_ref, ba1_ref, wa2_ref, ba2_ref,
                wc1_ref, bc1_ref, wc2_ref, bc2_ref,
                mu_ref, val_ref, sums_ref, cnts_ref):
    i = pl.program_id(0)

    @pl.when(i == 0)
    def _():
        sums_ref[...] = jnp.zeros_like(sums_ref)
        cnts_ref[...] = jnp.zeros_like(cnts_ref)

    a = acc_ref[0] + acc_ref[1]
    c = cnt_ref[0, :, :1] + cnt_ref[1, :, :1]
    mean = a / jnp.maximum(c, 1.0)
    h = jnp.maximum(
        jnp.dot(mean, wl_ref[...], preferred_element_type=jnp.float32)
        + b_ref[...]
        + jnp.dot(h1_ref[0], wr_ref[...], preferred_element_type=jnp.float32),
        0.0)
    oh = (batch_ref[...] == lax.broadcasted_iota(jnp.int32, (G, BLK), 0)
          ).astype(jnp.float32)
    sums_ref[...] += jnp.dot(oh, h, preferred_element_type=jnp.float32,
                             precision=lax.Precision.HIGHEST)
    cnts_ref[...] += jnp.sum(oh, axis=1, keepdims=True)

    @pl.when(i == pl.num_programs(0) - 1)
    def _():
        pooled = sums_ref[...] / jnp.maximum(cnts_ref[...], 1.0)
        ha = jnp.maximum(
            jnp.dot(pooled, wa1_ref[...], preferred_element_type=jnp.float32)
            + ba1_ref[...], 0.0)
        mu_ref[...] = (jnp.dot(ha, wa2_ref[...], preferred_element_type=jnp.float32)
                       + ba2_ref[...])
        hc = jnp.maximum(
            jnp.dot(pooled, wc1_ref[...], preferred_element_type=jnp.float32)
            + bc1_ref[...], 0.0)
        val_ref[...] = (jnp.dot(hc, wc2_ref[...], preferred_element_type=jnp.float32)
                        + bc2_ref[...])


def _final(acc, cnt, h1, wlT, wrT, b2d, batch_row,
           wa1T, ba1, wa2T, ba2, wc1T, bc1, wc2T, bc2):
    A = wa2T.shape[1]
    const = lambda shape: pl.BlockSpec(shape, lambda i: tuple(0 for _ in shape))
    return pl.pallas_call(
        _final_body,
        grid=(N_PAD // BLK,),
        in_specs=[
            pl.BlockSpec((NC, BLK, H), lambda i: (0, i, 0)),
            pl.BlockSpec((NC, BLK, CW), lambda i: (0, i, 0)),
            pl.BlockSpec((1, BLK, H), lambda i: (0, i, 0)),
            const((H, H)), const((H, H)), const((1, H)),
            pl.BlockSpec((1, BLK), lambda i: (0, i)),
            const((H, H)), const((1, H)), const((H, A)), const((1, A)),
            const((H, H)), const((1, H)), const((H, 1)), const((1, 1)),
        ],
        out_specs=[
            pl.BlockSpec((G, A), lambda i: (0, 0)),
            pl.BlockSpec((G, 1), lambda i: (0, 0)),
        ],
        out_shape=[
            jax.ShapeDtypeStruct((G, A), jnp.float32),
            jax.ShapeDtypeStruct((G, 1), jnp.float32),
        ],
        scratch_shapes=[
            pltpu.VMEM((G, H), jnp.float32),
            pltpu.VMEM((G, 1), jnp.float32),
        ],
    )(acc, cnt, h1, wlT, wrT, b2d, batch_row, wa1T, ba1, wa2T, ba2, wc1T, bc1, wc2T, bc2)


def kernel(x, edge_index, batch, W1l, b1l, W1r, W2l, b2l, W2r,
           Wa1, ba1, Wa2, ba2, Wc1, bc1, Wc2, bc2):
    E = edge_index.shape[1]

    def pad_edges(nparts):
        nch = -(-E // (nparts * CHUNK))
        nch = -(-nch // NBUF) * NBUF
        e_pad = nparts * nch * CHUNK
        ps = jnp.full((e_pad - E,), N, jnp.int32)
        pd = N + jnp.arange(e_pad - E, dtype=jnp.int32) % (N_PAD - N)
        s3 = jnp.concatenate([edge_index[0], ps]).reshape(nparts, nch, CHUNK)
        d3 = jnp.concatenate([edge_index[1], pd]).reshape(nparts, nch, CHUNK)
        return s3, d3, nch

    src1, dst1, nch1 = pad_edges(NS)
    src2, dst2, nch2 = pad_edges(NW)
    x_p = jnp.pad(x, ((0, N_PAD - N), (0, 0)))
    x2 = jnp.stack([x_p[:, :H], x_p[:, H:]])
    batch_row = jnp.pad(batch, (0, N_PAD - N), constant_values=G).reshape(1, N_PAD)

    zH = jnp.zeros((N_PAD, H), jnp.float32)
    z16 = jnp.zeros((N_PAD, CW), jnp.float32)
    ones = jnp.ones((CHUNK, CW), jnp.float32)

    acc1, cnt = _seg_sum_l1(x2, src1, dst1, zH, z16, ones, nch1)
    h1 = _layer1(acc1, cnt, x_p, W1l.T, W1r.T, b1l.reshape(1, H))
    acc2 = _seg_sum_nocnt(h1, src2, dst2, zH, nch2, H)
    mu, value = _final(acc2, cnt, h1, W2l.T, W2r.T, b2l.reshape(1, H), batch_row,
                       Wa1.T, ba1.reshape(1, -1), Wa2.T, ba2.reshape(1, -1),
                       Wc1.T, bc1.reshape(1, -1), Wc2.T, bc2.reshape(1, -1))
    return (mu, value)


# speedup vs baseline: 1.7714x; 1.7714x over previous
import functools

import jax
import jax.numpy as jnp
from jax import lax
from jax.experimental import pallas as pl
from jax.experimental.pallas import tpu as pltpu
from jax.experimental.pallas import tpu_sc as plsc

N = 10000
F = 128
H = 64
G = 64

N_PAD = 10240
BLK = 512
CHUNK = 128
CW = 16
NC = 2
NS = 16
NW = NC * NS
RPT = N_PAD // NS


NBUF = 2


def _seg_body_l1(nchunks, p2_hbm, src_hbm, dst_hbm, zh, z16, ones_hbm,
                 out_acc, out_cnt,
                 idx_s, idx_d, rows_v, ones_v, acc_sh, cnt_sh, gsem, ssem, csem):
    cid = lax.axis_index("c")
    sid = lax.axis_index("s")
    base = sid * RPT
    chalf = nchunks // 2
    pltpu.sync_copy(zh.at[pl.ds(base, RPT)], acc_sh.at[pl.ds(base, RPT)])
    pltpu.sync_copy(z16.at[pl.ds(base, RPT)], cnt_sh.at[pl.ds(base, RPT)])
    pltpu.sync_copy(src_hbm.at[sid], idx_s)
    pltpu.sync_copy(dst_hbm.at[sid], idx_d)
    pltpu.sync_copy(ones_hbm, ones_v)
    plsc.subcore_barrier()

    ptab = p2_hbm.at[cid]
    pltpu.async_copy(ptab.at[idx_s.at[0]], rows_v.at[0], gsem.at[0])

    def group(g, carry):
        for b in range(NBUF):
            j = g * NBUF + b
            ob = 1 - b

            @pl.when(j >= 1)
            def _():
                pltpu.make_async_copy(rows_v.at[ob], acc_sh.at[idx_d.at[0]],
                                      ssem.at[ob]).wait()

            @pl.when(j + 1 < nchunks)
            def _():
                pltpu.async_copy(ptab.at[idx_s.at[j + 1]], rows_v.at[ob],
                                 gsem.at[ob])

            pltpu.make_async_copy(ptab.at[idx_s.at[j]], rows_v.at[b],
                                  gsem.at[b]).wait()
            pltpu.async_copy(rows_v.at[b], acc_sh.at[idx_d.at[j]],
                             ssem.at[b], add=True)
            mine = ((cid == 0) & (j < chalf)) | ((cid == 1) & (j >= chalf))

            @pl.when(mine)
            def _():
                pltpu.async_copy(ones_v, cnt_sh.at[idx_d.at[j]], csem, add=True)
        return carry

    lax.fori_loop(0, nchunks // NBUF, group, 0)
    b_last = (nchunks - 1) % NBUF
    pltpu.make_async_copy(rows_v.at[b_last], acc_sh.at[idx_d.at[0]],
                          ssem.at[b_last]).wait()

    def drain(j, carry):
        pltpu.make_async_copy(ones_v, cnt_sh.at[idx_d.at[0]], csem).wait()
        return carry

    lax.fori_loop(0, chalf, drain, 0)
    plsc.subcore_barrier()
    pltpu.sync_copy(acc_sh.at[pl.ds(base, RPT)], out_acc.at[cid, pl.ds(base, RPT)])
    pltpu.sync_copy(cnt_sh.at[pl.ds(base, RPT)], out_cnt.at[cid, pl.ds(base, RPT)])


def _seg_body_nocnt(nchunks, p_hbm, src_hbm, dst_hbm, z64,
                    out_acc,
                    idx_s, idx_d, rows_v, acc_sh, gsem, ssem):
    cid = lax.axis_index("c")
    sid = lax.axis_index("s")
    wid = sid * NC + cid
    base = sid * RPT
    pltpu.sync_copy(z64.at[pl.ds(base, RPT)], acc_sh.at[pl.ds(base, RPT)])
    pltpu.sync_copy(src_hbm.at[wid], idx_s)
    pltpu.sync_copy(dst_hbm.at[wid], idx_d)
    plsc.subcore_barrier()

    ptab = p_hbm.at[cid]
    pltpu.async_copy(ptab.at[idx_s.at[0]], rows_v.at[0], gsem.at[0])

    def group(g, carry):
        for b in range(NBUF):
            j = g * NBUF + b
            ob = 1 - b

            @pl.when(j >= 1)
            def _():
                pltpu.make_async_copy(rows_v.at[ob], acc_sh.at[idx_d.at[j - 1]],
                                      ssem.at[ob]).wait()

            @pl.when(j + 1 < nchunks)
            def _():
                pltpu.async_copy(ptab.at[idx_s.at[j + 1]], rows_v.at[ob],
                                 gsem.at[ob])

            pltpu.make_async_copy(ptab.at[idx_s.at[j]], rows_v.at[b],
                                  gsem.at[b]).wait()
            pltpu.async_copy(rows_v.at[b], acc_sh.at[idx_d.at[j]],
                             ssem.at[b], add=True)
        return carry

    lax.fori_loop(0, nchunks // NBUF, group, 0)
    b_last = (nchunks - 1) % NBUF
    pltpu.make_async_copy(rows_v.at[b_last], acc_sh.at[idx_d.at[nchunks - 1]],
                          ssem.at[b_last]).wait()
    plsc.subcore_barrier()
    pltpu.sync_copy(acc_sh.at[pl.ds(base, RPT)], out_acc.at[cid, pl.ds(base, RPT)])


def _seg_sum_l1(x2, src2, dst2, zh, z16, ones, nchunks):
    mesh = plsc.VectorSubcoreMesh(core_axis_name="c", subcore_axis_name="s")
    kern = pl.kernel(
        functools.partial(_seg_body_l1, nchunks),
        mesh=mesh,
        compiler_params=pltpu.CompilerParams(use_tc_tiling_on_sc=False),
        out_type=[
            jax.ShapeDtypeStruct((NC, N_PAD, H), jnp.float32),
            jax.ShapeDtypeStruct((NC, N_PAD, CW), jnp.float32),
        ],
        scratch_types=[
            pltpu.VMEM((nchunks, CHUNK), jnp.int32),
            pltpu.VMEM((nchunks, CHUNK), jnp.int32),
            pltpu.VMEM((NBUF, CHUNK, H), jnp.float32),
            pltpu.VMEM((CHUNK, CW), jnp.float32),
            pltpu.VMEM_SHARED((N_PAD, H), jnp.float32),
            pltpu.VMEM_SHARED((N_PAD, CW), jnp.float32),
            pltpu.SemaphoreType.DMA((NBUF,)),
            pltpu.SemaphoreType.DMA((NBUF,)),
            pltpu.SemaphoreType.DMA,
        ],
    )
    return kern(x2, src2, dst2, zh, z16, ones)


def _seg_sum_nocnt(p, src3, dst3, zacc, nchunks, width):
    mesh = plsc.VectorSubcoreMesh(core_axis_name="c", subcore_axis_name="s")
    kern = pl.kernel(
        functools.partial(_seg_body_nocnt, nchunks),
        mesh=mesh,
        compiler_params=pltpu.CompilerParams(use_tc_tiling_on_sc=False),
        out_type=jax.ShapeDtypeStruct((NC, N_PAD, width), jnp.float32),
        scratch_types=[
            pltpu.VMEM((nchunks, CHUNK), jnp.int32),
            pltpu.VMEM((nchunks, CHUNK), jnp.int32),
            pltpu.VMEM((NBUF, CHUNK, width), jnp.float32),
            pltpu.VMEM_SHARED((N_PAD, width), jnp.float32),
            pltpu.SemaphoreType.DMA((NBUF,)),
            pltpu.SemaphoreType.DMA((NBUF,)),
        ],
    )
    return kern(p, src3, dst3, zacc)


def _layer1_body(acc_ref, cnt_ref, x_ref, wl_ref, wr_ref, b_ref, h_ref):
    a = jnp.concatenate([acc_ref[0], acc_ref[1]], axis=1)
    c = cnt_ref[0, :, :1] + cnt_ref[1, :, :1]
    mean = a / jnp.maximum(c, 1.0)
    h = jnp.maximum(
        jnp.dot(mean, wl_ref[...], preferred_element_type=jnp.float32)
        + b_ref[...]
        + jnp.dot(x_ref[...], wr_ref[...], preferred_element_type=jnp.float32),
        0.0)
    h_ref[0] = h
    h_ref[1] = h


def _layer1(acc, cnt, x_p, wlT, wrT, b2d):
    return pl.pallas_call(
        _layer1_body,
        grid=(N_PAD // BLK,),
        in_specs=[
            pl.BlockSpec((NC, BLK, H), lambda i: (0, i, 0)),
            pl.BlockSpec((NC, BLK, CW), lambda i: (0, i, 0)),
            pl.BlockSpec((BLK, F), lambda i: (i, 0)),
            pl.BlockSpec((F, H), lambda i: (0, 0)),
            pl.BlockSpec((F, H), lambda i: (0, 0)),
            pl.BlockSpec((1, H), lambda i: (0, 0)),
        ],
        out_specs=pl.BlockSpec((NC, BLK, H), lambda i: (0, i, 0)),
        out_shape=jax.ShapeDtypeStruct((NC, N_PAD, H), jnp.float32),
    )(acc, cnt, x_p, wlT, wrT, b2d)


def _final_body(acc_ref, cnt_ref, h1_ref, wl_ref, wr_ref, b_ref, batch_ref,
                wa1_ref, ba1_ref, wa2_ref, ba2_ref,
                wc1_ref, bc1_ref, wc2_ref, bc2_ref,
                mu_ref, val_ref, sums_ref, cnts_ref):
    i = pl.program_id(0)

    @pl.when(i == 0)
    def _():
        sums_ref[...] = jnp.zeros_like(sums_ref)
        cnts_ref[...] = jnp.zeros_like(cnts_ref)

    a = acc_ref[0] + acc_ref[1]
    c = cnt_ref[0, :, :1] + cnt_ref[1, :, :1]
    mean = a / jnp.maximum(c, 1.0)
    h = jnp.maximum(
        jnp.dot(mean, wl_ref[...], preferred_element_type=jnp.float32)
        + b_ref[...]
        + jnp.dot(h1_ref[0], wr_ref[...], preferred_element_type=jnp.float32),
        0.0)
    oh = (batch_ref[...] == lax.broadcasted_iota(jnp.int32, (G, BLK), 0)
          ).astype(jnp.float32)
    sums_ref[...] += jnp.dot(oh, h, preferred_element_type=jnp.float32,
                             precision=lax.Precision.HIGHEST)
    cnts_ref[...] += jnp.sum(oh, axis=1, keepdims=True)

    @pl.when(i == pl.num_programs(0) - 1)
    def _():
        pooled = sums_ref[...] / jnp.maximum(cnts_ref[...], 1.0)
        ha = jnp.maximum(
            jnp.dot(pooled, wa1_ref[...], preferred_element_type=jnp.float32)
            + ba1_ref[...], 0.0)
        mu_ref[...] = (jnp.dot(ha, wa2_ref[...], preferred_element_type=jnp.float32)
                       + ba2_ref[...])
        hc = jnp.maximum(
            jnp.dot(pooled, wc1_ref[...], preferred_element_type=jnp.float32)
            + bc1_ref[...], 0.0)
        val_ref[...] = (jnp.dot(hc, wc2_ref[...], preferred_element_type=jnp.float32)
                        + bc2_ref[...])


def _final(acc, cnt, h1, wlT, wrT, b2d, batch_row,
           wa1T, ba1, wa2T, ba2, wc1T, bc1, wc2T, bc2):
    A = wa2T.shape[1]
    const = lambda shape: pl.BlockSpec(shape, lambda i: tuple(0 for _ in shape))
    return pl.pallas_call(
        _final_body,
        grid=(N_PAD // BLK,),
        in_specs=[
            pl.BlockSpec((NC, BLK, H), lambda i: (0, i, 0)),
            pl.BlockSpec((NC, BLK, CW), lambda i: (0, i, 0)),
            pl.BlockSpec((1, BLK, H), lambda i: (0, i, 0)),
            const((H, H)), const((H, H)), const((1, H)),
            pl.BlockSpec((1, BLK), lambda i: (0, i)),
            const((H, H)), const((1, H)), const((H, A)), const((1, A)),
            const((H, H)), const((1, H)), const((H, 1)), const((1, 1)),
        ],
        out_specs=[
            pl.BlockSpec((G, A), lambda i: (0, 0)),
            pl.BlockSpec((G, 1), lambda i: (0, 0)),
        ],
        out_shape=[
            jax.ShapeDtypeStruct((G, A), jnp.float32),
            jax.ShapeDtypeStruct((G, 1), jnp.float32),
        ],
        scratch_shapes=[
            pltpu.VMEM((G, H), jnp.float32),
            pltpu.VMEM((G, 1), jnp.float32),
        ],
    )(acc, cnt, h1, wlT, wrT, b2d, batch_row, wa1T, ba1, wa2T, ba2, wc1T, bc1, wc2T, bc2)


def kernel(x, edge_index, batch, W1l, b1l, W1r, W2l, b2l, W2r,
           Wa1, ba1, Wa2, ba2, Wc1, bc1, Wc2, bc2):
    E = edge_index.shape[1]

    def pad_edges(nparts):
        nch = -(-E // (nparts * CHUNK))
        nch = -(-nch // NBUF) * NBUF
        per = nch * CHUNK
        er = -(-E // nparts)
        spread = N + jnp.arange(nparts * per - E, dtype=jnp.int32) % (N_PAD - N)

        def shard(e):
            flat = jnp.concatenate([e, spread[: nparts * er - E]])
            body = flat.reshape(nparts, er)
            tail = spread[nparts * er - E:].reshape(nparts, per - er)
            return jnp.concatenate([body, tail], axis=1).reshape(nparts, nch, CHUNK)

        return shard(edge_index[0]), shard(edge_index[1]), nch

    src1, dst1, nch1 = pad_edges(NS)
    src2, dst2, nch2 = pad_edges(NW)
    x_p = jnp.pad(x, ((0, N_PAD - N), (0, 0)))
    x2 = jnp.stack([x_p[:, :H], x_p[:, H:]])
    batch_row = jnp.pad(batch, (0, N_PAD - N), constant_values=G).reshape(1, N_PAD)

    zH = jnp.zeros((N_PAD, H), jnp.float32)
    z16 = jnp.zeros((N_PAD, CW), jnp.float32)
    ones = jnp.ones((CHUNK, CW), jnp.float32)

    acc1, cnt = _seg_sum_l1(x2, src1, dst1, zH, z16, ones, nch1)
    h1 = _layer1(acc1, cnt, x_p, W1l.T, W1r.T, b1l.reshape(1, H))
    acc2 = _seg_sum_nocnt(h1, src2, dst2, zH, nch2, H)
    mu, value = _final(acc2, cnt, h1, W2l.T, W2r.T, b2l.reshape(1, H), batch_row,
                       Wa1.T, ba1.reshape(1, -1), Wa2.T, ba2.reshape(1, -1),
                       Wc1.T, bc1.reshape(1, -1), Wc2.T, bc2.reshape(1, -1))
    return (mu, value)


# speedup vs baseline: 1.8270x; 1.0314x over previous
import functools

import jax
import jax.numpy as jnp
from jax import lax
from jax.experimental import pallas as pl
from jax.experimental.pallas import tpu as pltpu
from jax.experimental.pallas import tpu_sc as plsc

N = 10000
F = 128
H = 64
G = 64

N_PAD = 10240
BLK = 512
CHUNK = 128
CW = 16
NC = 2
NS = 16
NW = NC * NS
RPT = N_PAD // NS


NBUF = 4


def _seg_body_l1(nchunks, p2_hbm, src_hbm, dst_hbm, zh, z16, ones_hbm,
                 out_acc, out_cnt,
                 idx_s, idx_d, rows_v, ones_v, acc_sh, cnt_sh, gsem, ssem, csem):
    cid = lax.axis_index("c")
    sid = lax.axis_index("s")
    base = sid * RPT
    chalf = nchunks // 2
    pltpu.sync_copy(zh.at[pl.ds(base, RPT)], acc_sh.at[pl.ds(base, RPT)])
    pltpu.sync_copy(z16.at[pl.ds(base, RPT)], cnt_sh.at[pl.ds(base, RPT)])
    pltpu.sync_copy(src_hbm.at[sid], idx_s)
    pltpu.sync_copy(dst_hbm.at[sid], idx_d)
    pltpu.sync_copy(ones_hbm, ones_v)
    plsc.subcore_barrier()

    ptab = p2_hbm.at[cid]
    pltpu.async_copy(ptab.at[idx_s.at[0]], rows_v.at[0], gsem.at[0])
    pltpu.async_copy(ptab.at[idx_s.at[1]], rows_v.at[1], gsem.at[1])

    def group(g, carry):
        for b in range(NBUF):
            j = g * NBUF + b
            bg = (b + 2) % NBUF

            pltpu.make_async_copy(ptab.at[idx_s.at[j]], rows_v.at[b],
                                  gsem.at[b]).wait()
            pltpu.async_copy(rows_v.at[b], acc_sh.at[idx_d.at[j]],
                             ssem.at[b], add=True)
            mine = ((cid == 0) & (j < chalf)) | ((cid == 1) & (j >= chalf))

            @pl.when(mine)
            def _():
                pltpu.async_copy(ones_v, cnt_sh.at[idx_d.at[j]], csem, add=True)

            @pl.when(j >= 2)
            def _():
                pltpu.make_async_copy(rows_v.at[bg], acc_sh.at[idx_d.at[0]],
                                      ssem.at[bg]).wait()

            @pl.when(j + 2 < nchunks)
            def _():
                pltpu.async_copy(ptab.at[idx_s.at[j + 2]], rows_v.at[bg],
                                 gsem.at[bg])
        return carry

    lax.fori_loop(0, nchunks // NBUF, group, 0)
    for jt in (nchunks - 2, nchunks - 1):
        bt = jt % NBUF
        pltpu.make_async_copy(rows_v.at[bt], acc_sh.at[idx_d.at[0]],
                              ssem.at[bt]).wait()

    def drain(j, carry):
        pltpu.make_async_copy(ones_v, cnt_sh.at[idx_d.at[0]], csem).wait()
        return carry

    lax.fori_loop(0, chalf, drain, 0)
    plsc.subcore_barrier()
    pltpu.sync_copy(acc_sh.at[pl.ds(base, RPT)], out_acc.at[cid, pl.ds(base, RPT)])
    pltpu.sync_copy(cnt_sh.at[pl.ds(base, RPT)], out_cnt.at[cid, pl.ds(base, RPT)])


def _seg_body_nocnt(nchunks, p_hbm, src_hbm, dst_hbm, z64,
                    out_acc,
                    idx_s, idx_d, rows_v, acc_sh, gsem, ssem):
    cid = lax.axis_index("c")
    sid = lax.axis_index("s")
    wid = sid * NC + cid
    base = sid * RPT
    pltpu.sync_copy(z64.at[pl.ds(base, RPT)], acc_sh.at[pl.ds(base, RPT)])
    pltpu.sync_copy(src_hbm.at[wid], idx_s)
    pltpu.sync_copy(dst_hbm.at[wid], idx_d)
    plsc.subcore_barrier()

    ptab = p_hbm.at[cid]
    pltpu.async_copy(ptab.at[idx_s.at[0]], rows_v.at[0], gsem.at[0])
    pltpu.async_copy(ptab.at[idx_s.at[1]], rows_v.at[1], gsem.at[1])

    def group(g, carry):
        for b in range(NBUF):
            j = g * NBUF + b
            bg = (b + 2) % NBUF

            pltpu.make_async_copy(ptab.at[idx_s.at[j]], rows_v.at[b],
                                  gsem.at[b]).wait()
            pltpu.async_copy(rows_v.at[b], acc_sh.at[idx_d.at[j]],
                             ssem.at[b], add=True)

            @pl.when(j >= 2)
            def _():
                pltpu.make_async_copy(rows_v.at[bg], acc_sh.at[idx_d.at[0]],
                                      ssem.at[bg]).wait()

            @pl.when(j + 2 < nchunks)
            def _():
                pltpu.async_copy(ptab.at[idx_s.at[j + 2]], rows_v.at[bg],
                                 gsem.at[bg])
        return carry

    lax.fori_loop(0, nchunks // NBUF, group, 0)
    for jt in (nchunks - 2, nchunks - 1):
        bt = jt % NBUF
        pltpu.make_async_copy(rows_v.at[bt], acc_sh.at[idx_d.at[0]],
                              ssem.at[bt]).wait()
    plsc.subcore_barrier()
    pltpu.sync_copy(acc_sh.at[pl.ds(base, RPT)], out_acc.at[cid, pl.ds(base, RPT)])


def _seg_sum_l1(x2, src2, dst2, zh, z16, ones, nchunks):
    mesh = plsc.VectorSubcoreMesh(core_axis_name="c", subcore_axis_name="s")
    kern = pl.kernel(
        functools.partial(_seg_body_l1, nchunks),
        mesh=mesh,
        compiler_params=pltpu.CompilerParams(use_tc_tiling_on_sc=False),
        out_type=[
            jax.ShapeDtypeStruct((NC, N_PAD, H), jnp.float32),
            jax.ShapeDtypeStruct((NC, N_PAD, CW), jnp.float32),
        ],
        scratch_types=[
            pltpu.VMEM((nchunks, CHUNK), jnp.int32),
            pltpu.VMEM((nchunks, CHUNK), jnp.int32),
            pltpu.VMEM((NBUF, CHUNK, H), jnp.float32),
            pltpu.VMEM((CHUNK, CW), jnp.float32),
            pltpu.VMEM_SHARED((N_PAD, H), jnp.float32),
            pltpu.VMEM_SHARED((N_PAD, CW), jnp.float32),
            pltpu.SemaphoreType.DMA((NBUF,)),
            pltpu.SemaphoreType.DMA((NBUF,)),
            pltpu.SemaphoreType.DMA,
        ],
    )
    return kern(x2, src2, dst2, zh, z16, ones)


def _seg_sum_nocnt(p, src3, dst3, zacc, nchunks, width):
    mesh = plsc.VectorSubcoreMesh(core_axis_name="c", subcore_axis_name="s")
    kern = pl.kernel(
        functools.partial(_seg_body_nocnt, nchunks),
        mesh=mesh,
        compiler_params=pltpu.CompilerParams(use_tc_tiling_on_sc=False),
        out_type=jax.ShapeDtypeStruct((NC, N_PAD, width), jnp.float32),
        scratch_types=[
            pltpu.VMEM((nchunks, CHUNK), jnp.int32),
            pltpu.VMEM((nchunks, CHUNK), jnp.int32),
            pltpu.VMEM((NBUF, CHUNK, width), jnp.float32),
            pltpu.VMEM_SHARED((N_PAD, width), jnp.float32),
            pltpu.SemaphoreType.DMA((NBUF,)),
            pltpu.SemaphoreType.DMA((NBUF,)),
        ],
    )
    return kern(p, src3, dst3, zacc)


def _layer1_body(acc_ref, cnt_ref, x_ref, wl_ref, wr_ref, b_ref, h_ref):
    a = jnp.concatenate([acc_ref[0], acc_ref[1]], axis=1)
    c = cnt_ref[0, :, :1] + cnt_ref[1, :, :1]
    mean = a / jnp.maximum(c, 1.0)
    h = jnp.maximum(
        jnp.dot(mean, wl_ref[...], preferred_element_type=jnp.float32)
        + b_ref[...]
        + jnp.dot(x_ref[...], wr_ref[...], preferred_element_type=jnp.float32),
        0.0)
    h_ref[0] = h
    h_ref[1] = h


def _layer1(acc, cnt, x_p, wlT, wrT, b2d):
    return pl.pallas_call(
        _layer1_body,
        grid=(N_PAD // BLK,),
        in_specs=[
            pl.BlockSpec((NC, BLK, H), lambda i: (0, i, 0)),
            pl.BlockSpec((NC, BLK, CW), lambda i: (0, i, 0)),
            pl.BlockSpec((BLK, F), lambda i: (i, 0)),
            pl.BlockSpec((F, H), lambda i: (0, 0)),
            pl.BlockSpec((F, H), lambda i: (0, 0)),
            pl.BlockSpec((1, H), lambda i: (0, 0)),
        ],
        out_specs=pl.BlockSpec((NC, BLK, H), lambda i: (0, i, 0)),
        out_shape=jax.ShapeDtypeStruct((NC, N_PAD, H), jnp.float32),
    )(acc, cnt, x_p, wlT, wrT, b2d)


def _final_body(acc_ref, cnt_ref, h1_ref, wl_ref, wr_ref, b_ref, batch_ref,
                wa1_ref, ba1_ref, wa2_ref, ba2_ref,
                wc1_ref, bc1_ref, wc2_ref, bc2_ref,
                mu_ref, val_ref, sums_ref, cnts_ref):
    i = pl.program_id(0)

    @pl.when(i == 0)
    def _():
        sums_ref[...] = jnp.zeros_like(sums_ref)
        cnts_ref[...] = jnp.zeros_like(cnts_ref)

    a = acc_ref[0] + acc_ref[1]
    c = cnt_ref[0, :, :1] + cnt_ref[1, :, :1]
    mean = a / jnp.maximum(c, 1.0)
    h = jnp.maximum(
        jnp.dot(mean, wl_ref[...], preferred_element_type=jnp.float32)
        + b_ref[...]
        + jnp.dot(h1_ref[0], wr_ref[...], preferred_element_type=jnp.float32),
        0.0)
    oh = (batch_ref[...] == lax.broadcasted_iota(jnp.int32, (G, BLK), 0)
          ).astype(jnp.float32)
    sums_ref[...] += jnp.dot(oh, h, preferred_element_type=jnp.float32,
                             precision=lax.Precision.HIGHEST)
    cnts_ref[...] += jnp.sum(oh, axis=1, keepdims=True)

    @pl.when(i == pl.num_programs(0) - 1)
    def _():
        pooled = sums_ref[...] / jnp.maximum(cnts_ref[...], 1.0)
        ha = jnp.maximum(
            jnp.dot(pooled, wa1_ref[...], preferred_element_type=jnp.float32)
            + ba1_ref[...], 0.0)
        mu_ref[...] = (jnp.dot(ha, wa2_ref[...], preferred_element_type=jnp.float32)
                       + ba2_ref[...])
        hc = jnp.maximum(
            jnp.dot(pooled, wc1_ref[...], preferred_element_type=jnp.float32)
            + bc1_ref[...], 0.0)
        val_ref[...] = (jnp.dot(hc, wc2_ref[...], preferred_element_type=jnp.float32)
                        + bc2_ref[...])


def _final(acc, cnt, h1, wlT, wrT, b2d, batch_row,
           wa1T, ba1, wa2T, ba2, wc1T, bc1, wc2T, bc2):
    A = wa2T.shape[1]
    const = lambda shape: pl.BlockSpec(shape, lambda i: tuple(0 for _ in shape))
    return pl.pallas_call(
        _final_body,
        grid=(N_PAD // BLK,),
        in_specs=[
            pl.BlockSpec((NC, BLK, H), lambda i: (0, i, 0)),
            pl.BlockSpec((NC, BLK, CW), lambda i: (0, i, 0)),
            pl.BlockSpec((1, BLK, H), lambda i: (0, i, 0)),
            const((H, H)), const((H, H)), const((1, H)),
            pl.BlockSpec((1, BLK), lambda i: (0, i)),
            const((H, H)), const((1, H)), const((H, A)), const((1, A)),
            const((H, H)), const((1, H)), const((H, 1)), const((1, 1)),
        ],
        out_specs=[
            pl.BlockSpec((G, A), lambda i: (0, 0)),
            pl.BlockSpec((G, 1), lambda i: (0, 0)),
        ],
        out_shape=[
            jax.ShapeDtypeStruct((G, A), jnp.float32),
            jax.ShapeDtypeStruct((G, 1), jnp.float32),
        ],
        scratch_shapes=[
            pltpu.VMEM((G, H), jnp.float32),
            pltpu.VMEM((G, 1), jnp.float32),
        ],
    )(acc, cnt, h1, wlT, wrT, b2d, batch_row, wa1T, ba1, wa2T, ba2, wc1T, bc1, wc2T, bc2)


def kernel(x, edge_index, batch, W1l, b1l, W1r, W2l, b2l, W2r,
           Wa1, ba1, Wa2, ba2, Wc1, bc1, Wc2, bc2):
    E = edge_index.shape[1]

    def pad_edges(nparts):
        nch = -(-E // (nparts * CHUNK))
        nch = -(-nch // NBUF) * NBUF
        per = nch * CHUNK
        er = -(-E // nparts)
        spread = N + jnp.arange(nparts * per - E, dtype=jnp.int32) % (N_PAD - N)

        def shard(e):
            flat = jnp.concatenate([e, spread[: nparts * er - E]])
            body = flat.reshape(nparts, er)
            tail = spread[nparts * er - E:].reshape(nparts, per - er)
            return jnp.concatenate([body, tail], axis=1).reshape(nparts, nch, CHUNK)

        return shard(edge_index[0]), shard(edge_index[1]), nch

    src1, dst1, nch1 = pad_edges(NS)
    src2, dst2, nch2 = pad_edges(NW)
    x_p = jnp.pad(x, ((0, N_PAD - N), (0, 0)))
    x2 = jnp.stack([x_p[:, :H], x_p[:, H:]])
    batch_row = jnp.pad(batch, (0, N_PAD - N), constant_values=G).reshape(1, N_PAD)

    zH = jnp.zeros((N_PAD, H), jnp.float32)
    z16 = jnp.zeros((N_PAD, CW), jnp.float32)
    ones = jnp.ones((CHUNK, CW), jnp.float32)

    acc1, cnt = _seg_sum_l1(x2, src1, dst1, zH, z16, ones, nch1)
    h1 = _layer1(acc1, cnt, x_p, W1l.T, W1r.T, b1l.reshape(1, H))
    acc2 = _seg_sum_nocnt(h1, src2, dst2, zH, nch2, H)
    mu, value = _final(acc2, cnt, h1, W2l.T, W2r.T, b2l.reshape(1, H), batch_row,
                       Wa1.T, ba1.reshape(1, -1), Wa2.T, ba2.reshape(1, -1),
                       Wc1.T, bc1.reshape(1, -1), Wc2.T, bc2.reshape(1, -1))
    return (mu, value)


# speedup vs baseline: 1.8633x; 1.0199x over previous
import functools

import jax
import jax.numpy as jnp
from jax import lax
from jax.experimental import pallas as pl
from jax.experimental.pallas import tpu as pltpu
from jax.experimental.pallas import tpu_sc as plsc

N = 10000
F = 128
H = 64
G = 64

N_PAD = 10240
BLK = 512
CHUNK = 128
CW = 16
NC = 2
NS = 16
NW = NC * NS
RPT = N_PAD // NS


NBUF = 4


def _seg_body_l1(nchunks, p2_hbm, src_hbm, dst_hbm, zh, z16, ones_hbm,
                 out_acc, out_cnt,
                 idx_s, idx_d, rows_v, ones_v, acc_sh, cnt_sh, gsem, ssem, csem):
    cid = lax.axis_index("c")
    sid = lax.axis_index("s")
    base = sid * RPT
    chalf = nchunks // 2
    pltpu.sync_copy(zh.at[pl.ds(base, RPT)], acc_sh.at[pl.ds(base, RPT)])
    pltpu.sync_copy(z16.at[pl.ds(base, RPT)], cnt_sh.at[pl.ds(base, RPT)])
    pltpu.sync_copy(src_hbm.at[sid], idx_s)
    pltpu.sync_copy(dst_hbm.at[sid], idx_d)
    pltpu.sync_copy(ones_hbm, ones_v)
    plsc.subcore_barrier()

    ptab = p2_hbm.at[cid]
    pltpu.async_copy(ptab.at[idx_s.at[0]], rows_v.at[0], gsem.at[0])
    pltpu.async_copy(ptab.at[idx_s.at[1]], rows_v.at[1], gsem.at[1])

    def group(g, carry):
        for b in range(NBUF):
            j = g * NBUF + b
            bg = (b + 2) % NBUF

            pltpu.make_async_copy(ptab.at[idx_s.at[j]], rows_v.at[b],
                                  gsem.at[b]).wait()
            pltpu.async_copy(rows_v.at[b], acc_sh.at[idx_d.at[j]],
                             ssem.at[b], add=True)
            mine = ((cid == 0) & (j < chalf)) | ((cid == 1) & (j >= chalf))

            @pl.when(mine)
            def _():
                pltpu.async_copy(ones_v, cnt_sh.at[idx_d.at[j]], csem, add=True)

            @pl.when(j >= 2)
            def _():
                pltpu.make_async_copy(rows_v.at[bg], acc_sh.at[idx_d.at[0]],
                                      ssem.at[bg]).wait()

            @pl.when(j + 2 < nchunks)
            def _():
                pltpu.async_copy(ptab.at[idx_s.at[j + 2]], rows_v.at[bg],
                                 gsem.at[bg])
        return carry

    lax.fori_loop(0, nchunks // NBUF, group, 0)
    for jt in (nchunks - 2, nchunks - 1):
        bt = jt % NBUF
        pltpu.make_async_copy(rows_v.at[bt], acc_sh.at[idx_d.at[0]],
                              ssem.at[bt]).wait()

    def drain(j, carry):
        pltpu.make_async_copy(ones_v, cnt_sh.at[idx_d.at[0]], csem).wait()
        return carry

    lax.fori_loop(0, chalf, drain, 0)
    plsc.subcore_barrier()
    pltpu.sync_copy(acc_sh.at[pl.ds(base, RPT)], out_acc.at[cid, pl.ds(base, RPT)])
    pltpu.sync_copy(cnt_sh.at[pl.ds(base, RPT)], out_cnt.at[cid, pl.ds(base, RPT)])


def _seg_body_nocnt(nchunks, p_hbm, src_hbm, dst_hbm, z64,
                    out_acc,
                    idx_s, idx_d, rows_v, acc_sh, gsem, ssem):
    cid = lax.axis_index("c")
    sid = lax.axis_index("s")
    wid = sid * NC + cid
    base = sid * RPT
    pltpu.sync_copy(z64.at[pl.ds(base, RPT)], acc_sh.at[pl.ds(base, RPT)])
    pltpu.sync_copy(src_hbm.at[wid], idx_s)
    pltpu.sync_copy(dst_hbm.at[wid], idx_d)
    plsc.subcore_barrier()

    ptab = p_hbm
    pltpu.async_copy(ptab.at[idx_s.at[0]], rows_v.at[0], gsem.at[0])
    pltpu.async_copy(ptab.at[idx_s.at[1]], rows_v.at[1], gsem.at[1])

    def group(g, carry):
        for b in range(NBUF):
            j = g * NBUF + b
            bg = (b + 2) % NBUF

            pltpu.make_async_copy(ptab.at[idx_s.at[j]], rows_v.at[b],
                                  gsem.at[b]).wait()
            pltpu.async_copy(rows_v.at[b], acc_sh.at[idx_d.at[j]],
                             ssem.at[b], add=True)

            @pl.when(j >= 2)
            def _():
                pltpu.make_async_copy(rows_v.at[bg], acc_sh.at[idx_d.at[0]],
                                      ssem.at[bg]).wait()

            @pl.when(j + 2 < nchunks)
            def _():
                pltpu.async_copy(ptab.at[idx_s.at[j + 2]], rows_v.at[bg],
                                 gsem.at[bg])
        return carry

    lax.fori_loop(0, nchunks // NBUF, group, 0)
    for jt in (nchunks - 2, nchunks - 1):
        bt = jt % NBUF
        pltpu.make_async_copy(rows_v.at[bt], acc_sh.at[idx_d.at[0]],
                              ssem.at[bt]).wait()
    plsc.subcore_barrier()
    pltpu.sync_copy(acc_sh.at[pl.ds(base, RPT)], out_acc.at[cid, pl.ds(base, RPT)])


def _seg_sum_l1(x2, src2, dst2, zh, z16, ones, nchunks):
    mesh = plsc.VectorSubcoreMesh(core_axis_name="c", subcore_axis_name="s")
    kern = pl.kernel(
        functools.partial(_seg_body_l1, nchunks),
        mesh=mesh,
        compiler_params=pltpu.CompilerParams(use_tc_tiling_on_sc=False),
        out_type=[
            jax.ShapeDtypeStruct((NC, N_PAD, H), jnp.float32),
            jax.ShapeDtypeStruct((NC, N_PAD, CW), jnp.float32),
        ],
        scratch_types=[
            pltpu.VMEM((nchunks, CHUNK), jnp.int32),
            pltpu.VMEM((nchunks, CHUNK), jnp.int32),
            pltpu.VMEM((NBUF, CHUNK, H), jnp.float32),
            pltpu.VMEM((CHUNK, CW), jnp.float32),
            pltpu.VMEM_SHARED((N_PAD, H), jnp.float32),
            pltpu.VMEM_SHARED((N_PAD, CW), jnp.float32),
            pltpu.SemaphoreType.DMA((NBUF,)),
            pltpu.SemaphoreType.DMA((NBUF,)),
            pltpu.SemaphoreType.DMA,
        ],
    )
    return kern(x2, src2, dst2, zh, z16, ones)


def _seg_sum_nocnt(p, src3, dst3, zacc, nchunks, width):
    mesh = plsc.VectorSubcoreMesh(core_axis_name="c", subcore_axis_name="s")
    kern = pl.kernel(
        functools.partial(_seg_body_nocnt, nchunks),
        mesh=mesh,
        compiler_params=pltpu.CompilerParams(use_tc_tiling_on_sc=False),
        out_type=jax.ShapeDtypeStruct((NC, N_PAD, width), jnp.float32),
        scratch_types=[
            pltpu.VMEM((nchunks, CHUNK), jnp.int32),
            pltpu.VMEM((nchunks, CHUNK), jnp.int32),
            pltpu.VMEM((NBUF, CHUNK, width), jnp.float32),
            pltpu.VMEM_SHARED((N_PAD, width), jnp.float32),
            pltpu.SemaphoreType.DMA((NBUF,)),
            pltpu.SemaphoreType.DMA((NBUF,)),
        ],
    )
    return kern(p, src3, dst3, zacc)


def _layer1_body(acc_ref, cnt_ref, x_ref, wl_ref, wr_ref, b_ref, h_ref):
    a = jnp.concatenate([acc_ref[0], acc_ref[1]], axis=1)
    c = cnt_ref[0, :, :1] + cnt_ref[1, :, :1]
    mean = a / jnp.maximum(c, 1.0)
    h = jnp.maximum(
        jnp.dot(mean, wl_ref[...], preferred_element_type=jnp.float32)
        + b_ref[...]
        + jnp.dot(x_ref[...], wr_ref[...], preferred_element_type=jnp.float32),
        0.0)
    h_ref[...] = h


def _layer1(acc, cnt, x_p, wlT, wrT, b2d):
    return pl.pallas_call(
        _layer1_body,
        grid=(N_PAD // BLK,),
        in_specs=[
            pl.BlockSpec((NC, BLK, H), lambda i: (0, i, 0)),
            pl.BlockSpec((NC, BLK, CW), lambda i: (0, i, 0)),
            pl.BlockSpec((BLK, F), lambda i: (i, 0)),
            pl.BlockSpec((F, H), lambda i: (0, 0)),
            pl.BlockSpec((F, H), lambda i: (0, 0)),
            pl.BlockSpec((1, H), lambda i: (0, 0)),
        ],
        out_specs=pl.BlockSpec((BLK, H), lambda i: (i, 0)),
        out_shape=jax.ShapeDtypeStruct((N_PAD, H), jnp.float32),
    )(acc, cnt, x_p, wlT, wrT, b2d)


def _final_body(acc_ref, cnt_ref, h1_ref, wl_ref, wr_ref, b_ref, batch_ref,
                wa1_ref, ba1_ref, wa2_ref, ba2_ref,
                wc1_ref, bc1_ref, wc2_ref, bc2_ref,
                mu_ref, val_ref, sums_ref, cnts_ref):
    i = pl.program_id(0)

    @pl.when(i == 0)
    def _():
        sums_ref[...] = jnp.zeros_like(sums_ref)
        cnts_ref[...] = jnp.zeros_like(cnts_ref)

    a = acc_ref[0] + acc_ref[1]
    c = cnt_ref[0, :, :1] + cnt_ref[1, :, :1]
    mean = a / jnp.maximum(c, 1.0)
    h = jnp.maximum(
        jnp.dot(mean, wl_ref[...], preferred_element_type=jnp.float32)
        + b_ref[...]
        + jnp.dot(h1_ref[...], wr_ref[...], preferred_element_type=jnp.float32),
        0.0)
    oh = (batch_ref[...] == lax.broadcasted_iota(jnp.int32, (G, BLK), 0)
          ).astype(jnp.float32)
    sums_ref[...] += jnp.dot(oh, h, preferred_element_type=jnp.float32,
                             precision=lax.Precision.HIGHEST)
    cnts_ref[...] += jnp.sum(oh, axis=1, keepdims=True)

    @pl.when(i == pl.num_programs(0) - 1)
    def _():
        pooled = sums_ref[...] / jnp.maximum(cnts_ref[...], 1.0)
        ha = jnp.maximum(
            jnp.dot(pooled, wa1_ref[...], preferred_element_type=jnp.float32)
            + ba1_ref[...], 0.0)
        mu_ref[...] = (jnp.dot(ha, wa2_ref[...], preferred_element_type=jnp.float32)
                       + ba2_ref[...])
        hc = jnp.maximum(
            jnp.dot(pooled, wc1_ref[...], preferred_element_type=jnp.float32)
            + bc1_ref[...], 0.0)
        val_ref[...] = (jnp.dot(hc, wc2_ref[...], preferred_element_type=jnp.float32)
                        + bc2_ref[...])


def _final(acc, cnt, h1, wlT, wrT, b2d, batch_row,
           wa1T, ba1, wa2T, ba2, wc1T, bc1, wc2T, bc2):
    A = wa2T.shape[1]
    const = lambda shape: pl.BlockSpec(shape, lambda i: tuple(0 for _ in shape))
    return pl.pallas_call(
        _final_body,
        grid=(N_PAD // BLK,),
        in_specs=[
            pl.BlockSpec((NC, BLK, H), lambda i: (0, i, 0)),
            pl.BlockSpec((NC, BLK, CW), lambda i: (0, i, 0)),
            pl.BlockSpec((BLK, H), lambda i: (i, 0)),
            const((H, H)), const((H, H)), const((1, H)),
            pl.BlockSpec((1, BLK), lambda i: (0, i)),
            const((H, H)), const((1, H)), const((H, A)), const((1, A)),
            const((H, H)), const((1, H)), const((H, 1)), const((1, 1)),
        ],
        out_specs=[
            pl.BlockSpec((G, A), lambda i: (0, 0)),
            pl.BlockSpec((G, 1), lambda i: (0, 0)),
        ],
        out_shape=[
            jax.ShapeDtypeStruct((G, A), jnp.float32),
            jax.ShapeDtypeStruct((G, 1), jnp.float32),
        ],
        scratch_shapes=[
            pltpu.VMEM((G, H), jnp.float32),
            pltpu.VMEM((G, 1), jnp.float32),
        ],
    )(acc, cnt, h1, wlT, wrT, b2d, batch_row, wa1T, ba1, wa2T, ba2, wc1T, bc1, wc2T, bc2)


def kernel(x, edge_index, batch, W1l, b1l, W1r, W2l, b2l, W2r,
           Wa1, ba1, Wa2, ba2, Wc1, bc1, Wc2, bc2):
    E = edge_index.shape[1]

    def pad_edges(nparts):
        nch = -(-E // (nparts * CHUNK))
        nch = -(-nch // NBUF) * NBUF
        per = nch * CHUNK
        er = -(-E // nparts)
        spread = N + jnp.arange(nparts * per - E, dtype=jnp.int32) % (N_PAD - N)

        def shard(e):
            flat = jnp.concatenate([e, spread[: nparts * er - E]])
            body = flat.reshape(nparts, er)
            tail = spread[nparts * er - E:].reshape(nparts, per - er)
            return jnp.concatenate([body, tail], axis=1).reshape(nparts, nch, CHUNK)

        return shard(edge_index[0]), shard(edge_index[1]), nch

    src1, dst1, nch1 = pad_edges(NS)
    src2, dst2, nch2 = pad_edges(NW)
    x_p = jnp.pad(x, ((0, N_PAD - N), (0, 0)))
    x2 = jnp.stack([x_p[:, :H], x_p[:, H:]])
    batch_row = jnp.pad(batch, (0, N_PAD - N), constant_values=G).reshape(1, N_PAD)

    zH = jnp.zeros((N_PAD, H), jnp.float32)
    z16 = jnp.zeros((N_PAD, CW), jnp.float32)
    ones = jnp.ones((CHUNK, CW), jnp.float32)

    acc1, cnt = _seg_sum_l1(x2, src1, dst1, zH, z16, ones, nch1)
    h1 = _layer1(acc1, cnt, x_p, W1l.T, W1r.T, b1l.reshape(1, H))
    acc2 = _seg_sum_nocnt(h1, src2, dst2, zH, nch2, H)
    mu, value = _final(acc2, cnt, h1, W2l.T, W2r.T, b2l.reshape(1, H), batch_row,
                       Wa1.T, ba1.reshape(1, -1), Wa2.T, ba2.reshape(1, -1),
                       Wc1.T, bc1.reshape(1, -1), Wc2.T, bc2.reshape(1, -1))
    return (mu, value)
